# baseline (speedup 1.0000x reference)
import math

import jax
import jax.numpy as jnp
import numpy as np
from jax import lax
from jax.experimental import pallas as pl
from jax.experimental.pallas import tpu as pltpu
from jax.experimental.pallas import tpu_sc as plsc

_MAX_L = 4
_NL = 35


def _tables():
    lst = []
    for l in range(_MAX_L + 1):
        for lx in range(l, -1, -1):
            for ly in range(l - lx, -1, -1):
                lst.append((lx, ly, l - lx - ly))
    pref = np.zeros((_NL,), np.float64)
    for i, (lx, ly, lz) in enumerate(lst):
        l = lx + ly + lz
        if l == 0:
            continue
        pref[i] = math.factorial(l) / (
            math.factorial(lx) * math.factorial(ly) * math.factorial(lz))
    return pref


_PREF = _tables()
_SLOT_STARTS = (1, 4, 10, 20)


def _tc_body(p_ref, x_ref, o_ref):
    j = pl.program_id(1)
    l = j + 1
    x = x_ref[0, 0]
    pref = p_ref[0, j]
    t = (x * x) * pref
    first = (l == 1) | (l == 4) | (l == 10) | (l == 20)

    @pl.when(first)
    def _():
        o_ref[0, 0] = t

    @pl.when(jnp.logical_not(first))
    def _():
        o_ref[0, 0] += t


def _sc_body(x_hbm, o_hbm, x_v):
    wid = lax.axis_index("s") * 2 + lax.axis_index("c")

    @pl.when(wid < 8)
    def _():
        pltpu.sync_copy(x_hbm.at[wid, 0], x_v)
        pltpu.sync_copy(x_v, o_hbm.at[wid, 0])


def kernel(node_attr):
    N, R, L, C = node_attr.shape
    xt = jnp.transpose(node_attr, (1, 2, 3, 0))

    def _slot(j):
        l = j + 1
        return (l >= 4).astype(jnp.int32) + (l >= 10) + (l >= 20)

    pref_tab = jnp.asarray(_PREF[1:], jnp.float32).reshape(1, L - 1)
    yt = pl.pallas_call(
        _tc_body,
        grid=(R, L - 1),
        in_specs=[
            pl.BlockSpec(memory_space=pltpu.SMEM),
            pl.BlockSpec((1, 1, C, N), lambda r, j: (r, j + 1, 0, 0)),
        ],
        out_specs=pl.BlockSpec((1, 1, C, N), lambda r, j: (r, 1 + _slot(j), 0, 0)),
        out_shape=jax.ShapeDtypeStruct((R, 5, C, N), jnp.float32),
    )(pref_tab, xt)

    mesh = plsc.VectorSubcoreMesh(core_axis_name="c", subcore_axis_name="s")
    sc_run = pl.kernel(
        _sc_body,
        out_type=jax.ShapeDtypeStruct((R, 1, C, N), jnp.float32),
        mesh=mesh,
        scratch_types=[pltpu.VMEM((C, N), jnp.float32)],
        compiler_params=pltpu.CompilerParams(use_tc_tiling_on_sc=True),
    )
    y0 = sc_run(xt)

    yt = lax.dynamic_update_slice(yt, y0, (0, 0, 0, 0))
    return jnp.transpose(yt, (3, 0, 1, 2))


# speedup vs baseline: 5.2175x; 5.2175x over previous
import math

import jax
import jax.numpy as jnp
import numpy as np
from jax.experimental import pallas as pl


_MAX_L = 4
_NL = 35


def _tables():
    lst = []
    for l in range(_MAX_L + 1):
        for lx in range(l, -1, -1):
            for ly in range(l - lx, -1, -1):
                lst.append((lx, ly, l - lx - ly))
    pref = np.zeros((_NL,), np.float64)
    slot = np.full((_NL,), -1, np.int64)
    for i, (lx, ly, lz) in enumerate(lst):
        l = lx + ly + lz
        if l == 0:
            continue
        pref[i] = math.factorial(l) / (
            math.factorial(lx) * math.factorial(ly) * math.factorial(lz))
        slot[i] = l
    return pref, slot


_PREF, _SLOT = _tables()
_BLOCKS = {1: (1, 4), 2: (4, 10), 3: (10, 20), 4: (20, 35)}


def _body(x_ref, o_ref):
    for rr in range(x_ref.shape[0]):
        o_ref[rr, 0] = x_ref[rr, 0]
        for s, (lo, hi) in _BLOCKS.items():
            acc = None
            for l in range(lo, hi):
                x = x_ref[rr, l]
                t = (x * x) * jnp.float32(_PREF[l])
                acc = t if acc is None else acc + t
            o_ref[rr, s] = acc


def kernel(node_attr):
    N, R, L, C = node_attr.shape
    xt = jnp.transpose(node_attr, (1, 2, 3, 0))

    BR = 2
    grid = (R // BR,)
    yt = pl.pallas_call(
        _body,
        grid=grid,
        in_specs=[pl.BlockSpec((BR, L, C, N), lambda r: (r, 0, 0, 0))],
        out_specs=pl.BlockSpec((BR, 5, C, N), lambda r: (r, 0, 0, 0)),
        out_shape=jax.ShapeDtypeStruct((R, 5, C, N), jnp.float32),
    )(xt)
    return jnp.transpose(yt, (3, 0, 1, 2))


# speedup vs baseline: 5.3546x; 1.0263x over previous
import math

import jax
import jax.numpy as jnp
import numpy as np
from jax.experimental import pallas as pl


_MAX_L = 4
_NL = 35


def _tables():
    lst = []
    for l in range(_MAX_L + 1):
        for lx in range(l, -1, -1):
            for ly in range(l - lx, -1, -1):
                lst.append((lx, ly, l - lx - ly))
    pref = np.zeros((_NL,), np.float64)
    slot = np.full((_NL,), -1, np.int64)
    for i, (lx, ly, lz) in enumerate(lst):
        l = lx + ly + lz
        if l == 0:
            continue
        pref[i] = math.factorial(l) / (
            math.factorial(lx) * math.factorial(ly) * math.factorial(lz))
        slot[i] = l
    return pref, slot


_PREF, _SLOT = _tables()
_BLOCKS = {1: (1, 4), 2: (4, 10), 3: (10, 20), 4: (20, 35)}


def _body(x_ref, o_ref):
    for rr in range(x_ref.shape[0]):
        o_ref[rr, 0] = x_ref[rr, 0]
        for s, (lo, hi) in _BLOCKS.items():
            acc = None
            for l in range(lo, hi):
                x = x_ref[rr, l]
                t = (x * x) * jnp.float32(_PREF[l])
                acc = t if acc is None else acc + t
            o_ref[rr, s] = acc


def kernel(node_attr):
    N, R, L, C = node_attr.shape
    xt = jnp.transpose(node_attr, (1, 2, 3, 0))

    BR = 1
    grid = (R // BR,)
    yt = pl.pallas_call(
        _body,
        grid=grid,
        in_specs=[pl.BlockSpec((BR, L, C, N), lambda r: (r, 0, 0, 0))],
        out_specs=pl.BlockSpec((BR, 5, C, N), lambda r: (r, 0, 0, 0)),
        out_shape=jax.ShapeDtypeStruct((R, 5, C, N), jnp.float32),
    )(xt)
    return jnp.transpose(yt, (3, 0, 1, 2))
